# Initial kernel scaffold; baseline (speedup 1.0000x reference)
import jax
import jax.numpy as jnp
from jax.experimental import pallas as pl


def kernel(predicted_boxes, predicted_scores, target_boxes, target_labels, default_boxes):
    raise NotImplementedError("write your pallas kernel here")


# speedup vs baseline: 31.5665x; 31.5665x over previous
import functools

import jax
import jax.numpy as jnp
from jax import lax
from jax.experimental import pallas as pl

_LANES = 128
_INF_BITS = 0x7F800000


def _image_kernel(n_obj, n_classes, n_valid, sc_ref, pb_ref, tb_ref, tl_ref,
                  db_ref, neg_ref, npos_ref, hub_ref, pce_ref):
    S, L = db_ref.shape[1], db_ref.shape[2]

    fi = (lax.broadcasted_iota(jnp.int32, (S, L), 0) * L
          + lax.broadcasted_iota(jnp.int32, (S, L), 1))
    valid = fi < n_valid

    p_cx = db_ref[0]
    p_cy = db_ref[1]
    p_w = db_ref[2]
    p_h = db_ref[3]
    d_x1 = p_cx - p_w * 0.5
    d_y1 = p_cy - p_h * 0.5
    d_x2 = p_cx + p_w * 0.5
    d_y2 = p_cy + p_h * 0.5
    area_b = (d_x2 - d_x1) * (d_y2 - d_y1)

    best = jnp.full((S, L), -1.0, jnp.float32)
    best_idx = jnp.zeros((S, L), jnp.int32)
    box_idx = []
    for j in range(n_obj):
        cx = tb_ref[0, j, 0]
        cy = tb_ref[0, j, 1]
        w = tb_ref[0, j, 2]
        h = tb_ref[0, j, 3]
        ax1 = cx - w * 0.5
        ay1 = cy - h * 0.5
        ax2 = cx + w * 0.5
        ay2 = cy + h * 0.5
        area_a = (ax2 - ax1) * (ay2 - ay1)
        lt_x = jnp.maximum(ax1, d_x1)
        lt_y = jnp.maximum(ay1, d_y1)
        rb_x = jnp.minimum(ax2, d_x2)
        rb_y = jnp.minimum(ay2, d_y2)
        inter = (jnp.maximum(rb_x - lt_x, 0.0)
                 * jnp.maximum(rb_y - lt_y, 0.0))
        union = jnp.maximum(area_a + area_b - inter, 1e-10)
        iou = jnp.where(valid, inter / union, -1.0)
        upd = iou > best
        best = jnp.where(upd, iou, best)
        best_idx = jnp.where(upd, j, best_idx)
        m = jnp.max(iou)
        box_idx.append(jnp.min(jnp.where(iou == m, fi, jnp.int32(1 << 30))))

    for j in range(n_obj):
        best_idx = jnp.where(fi == box_idx[j], j, best_idx)

    check = best > 0.5

    g_lab = jnp.zeros((S, L), jnp.int32)
    g_cx = jnp.zeros((S, L), jnp.float32)
    g_cy = jnp.zeros((S, L), jnp.float32)
    g_w = jnp.zeros((S, L), jnp.float32)
    g_h = jnp.zeros((S, L), jnp.float32)
    for j in range(n_obj):
        eq = best_idx == j
        g_lab = jnp.where(eq, tl_ref[0, 0, j], g_lab)
        g_cx = jnp.where(eq, tb_ref[0, j, 0], g_cx)
        g_cy = jnp.where(eq, tb_ref[0, j, 1], g_cy)
        g_w = jnp.where(eq, tb_ref[0, j, 2], g_w)
        g_h = jnp.where(eq, tb_ref[0, j, 3], g_h)

    true_lab = jnp.where(check, g_lab, 0)
    positive = true_lab != 0
    t_cx = jnp.where(check, g_cx, 0.0)
    t_cy = jnp.where(check, g_cy, 0.0)
    t_w = jnp.where(check, g_w, 0.0)
    t_h = jnp.where(check, g_h, 0.0)

    e_x = (t_cx - p_cx) / (p_w * 0.1)
    e_y = (t_cy - p_cy) / (p_h * 0.1)
    e_w = jnp.log(jnp.maximum(t_w, 1e-8) / p_w) * 5.0
    e_h = jnp.log(jnp.maximum(t_h, 1e-8) / p_h) * 5.0

    hub = jnp.zeros((S, L), jnp.float32)
    for c, enc in enumerate((e_x, e_y, e_w, e_h)):
        diff = jnp.abs(pb_ref[0, c] - enc)
        hub = hub + jnp.where(diff < 1.0, 0.5 * diff * diff, diff - 0.5)
    hub_sum = jnp.sum(jnp.where(positive, hub, 0.0))

    sc = sc_ref[0]
    mx = jnp.max(sc, axis=0)
    ssum = jnp.sum(jnp.exp(sc - mx[None]), axis=0)
    lse = mx + jnp.log(ssum)
    sel = jnp.zeros((S, L), jnp.float32)
    for c in range(n_classes):
        sel = jnp.where(true_lab == c, sc[c], sel)
    ce = lse - sel

    n_pos = jnp.sum(jnp.where(positive, 1.0, 0.0))
    pce = jnp.sum(jnp.where(positive, ce, 0.0))
    neg = jnp.where(valid, jnp.where(positive, 0.0, ce), -1.0)

    neg_ref[0] = neg
    npos_ref[0] = jnp.full((1, L), n_pos, jnp.float32)
    hub_ref[0] = jnp.full((1, L), hub_sum, jnp.float32)
    pce_ref[0] = jnp.full((1, L), pce, jnp.float32)


def _mine_kernel(neg_ref, npos_ref, hub_ref, pce_ref, out_ref):
    neg = neg_ref[...]
    bits = lax.bitcast_convert_type(neg, jnp.int32)
    k = npos_ref[:, :, 0:1] * 3.0

    def body(_, carry):
        lo, hi = carry
        mid = lo + ((hi - lo + 1) >> 1)
        cnt = jnp.sum(jnp.where(bits >= mid, 1.0, 0.0), axis=(1, 2),
                      keepdims=True)
        pred = cnt >= k
        return (jnp.where(pred, mid, lo),
                jnp.where(pred, hi, mid - 1))

    B = neg.shape[0]
    lo0 = jnp.zeros((B, 1, 1), jnp.int32)
    hi0 = jnp.full((B, 1, 1), _INF_BITS, jnp.int32)
    lo, _ = lax.fori_loop(0, 31, body, (lo0, hi0))
    t = lax.bitcast_convert_type(lo, jnp.float32)

    gt = neg > t
    cnt_gt = jnp.sum(jnp.where(gt, 1.0, 0.0), axis=(1, 2), keepdims=True)
    sum_gt = jnp.sum(jnp.where(gt, neg, 0.0), axis=(1, 2), keepdims=True)
    hn_row = sum_gt + t * (k - cnt_gt)
    hn = jnp.sum(jnp.where(k > 0.0, hn_row, 0.0))

    n_pos = jnp.sum(npos_ref[:, :, 0:1])
    pos_ce = jnp.sum(pce_ref[:, :, 0:1])
    hub = jnp.sum(hub_ref[:, :, 0:1])
    loss = (pos_ce + hn) / n_pos + hub / (4.0 * n_pos)
    out_ref[...] = jnp.full((8, _LANES), loss, jnp.float32)


@jax.jit
def kernel(predicted_boxes, predicted_scores, target_boxes, target_labels,
           default_boxes):
    B, N, C = predicted_scores.shape
    n_obj = target_boxes.shape[1]
    L = _LANES
    S = -(-N // L)
    S = -(-S // 8) * 8
    pad = S * L - N

    ps = jnp.pad(jnp.transpose(predicted_scores, (0, 2, 1)),
                 ((0, 0), (0, 0), (0, pad))).reshape(B, C, S, L)
    pb = jnp.pad(jnp.transpose(predicted_boxes, (0, 2, 1)),
                 ((0, 0), (0, 0), (0, pad))).reshape(B, 4, S, L)
    db = jnp.pad(jnp.transpose(default_boxes, (1, 0)),
                 ((0, 0), (0, pad))).reshape(4, S, L)
    tb = target_boxes.astype(jnp.float32)
    tl = target_labels.astype(jnp.int32).reshape(B, 1, n_obj)

    f32 = jnp.float32
    neg, npos, hub, pce = pl.pallas_call(
        functools.partial(_image_kernel, n_obj, C, N),
        grid=(B,),
        in_specs=[
            pl.BlockSpec((1, C, S, L), lambda b: (b, 0, 0, 0)),
            pl.BlockSpec((1, 4, S, L), lambda b: (b, 0, 0, 0)),
            pl.BlockSpec((1, n_obj, 4), lambda b: (b, 0, 0)),
            pl.BlockSpec((1, 1, n_obj), lambda b: (b, 0, 0)),
            pl.BlockSpec((4, S, L), lambda b: (0, 0, 0)),
        ],
        out_specs=[
            pl.BlockSpec((1, S, L), lambda b: (b, 0, 0)),
            pl.BlockSpec((1, 1, L), lambda b: (b, 0, 0)),
            pl.BlockSpec((1, 1, L), lambda b: (b, 0, 0)),
            pl.BlockSpec((1, 1, L), lambda b: (b, 0, 0)),
        ],
        out_shape=[
            jax.ShapeDtypeStruct((B, S, L), f32),
            jax.ShapeDtypeStruct((B, 1, L), f32),
            jax.ShapeDtypeStruct((B, 1, L), f32),
            jax.ShapeDtypeStruct((B, 1, L), f32),
        ],
    )(ps, pb, tb, tl, db)

    out = pl.pallas_call(
        _mine_kernel,
        out_shape=jax.ShapeDtypeStruct((8, L), f32),
    )(neg, npos, hub, pce)
    return out[0, 0]


# speedup vs baseline: 77.9389x; 2.4690x over previous
import functools

import jax
import jax.numpy as jnp
from jax import lax
from jax.experimental import pallas as pl
from jax.experimental.pallas import tpu as pltpu

_LANES = 128
_INF_BITS = 0x7F800000


def _image_kernel(n_obj, n_classes, n_valid, ipb, sc_ref, pb_ref, tb_ref,
                  tl_ref, db_ref, neg_ref, npos_ref, hub_ref, pce_ref,
                  iou_scr):
    S, L = db_ref.shape[1], db_ref.shape[2]

    fi = (lax.broadcasted_iota(jnp.int32, (S, L), 0) * L
          + lax.broadcasted_iota(jnp.int32, (S, L), 1))
    valid = fi < n_valid

    p_cx = db_ref[0]
    p_cy = db_ref[1]
    p_w = db_ref[2]
    p_h = db_ref[3]
    d_x1 = p_cx - p_w * 0.5
    d_y1 = p_cy - p_h * 0.5
    d_x2 = p_cx + p_w * 0.5
    d_y2 = p_cy + p_h * 0.5
    area_b = (d_x2 - d_x1) * (d_y2 - d_y1)

    tbs = [[[tb_ref[i, j, k] for k in range(4)] for j in range(n_obj)]
           for i in range(ipb)]
    tls = [[tl_ref[i, 0, j] for j in range(n_obj)] for i in range(ipb)]

    bests = []
    best_idxs = []
    for i in range(ipb):
        best = jnp.full((S, L), -1.0, jnp.float32)
        best_idx = jnp.zeros((S, L), jnp.int32)
        for j in range(n_obj):
            cx, cy, w, h = tbs[i][j]
            ax1 = cx - w * 0.5
            ay1 = cy - h * 0.5
            ax2 = cx + w * 0.5
            ay2 = cy + h * 0.5
            area_a = (ax2 - ax1) * (ay2 - ay1)
            inter = (jnp.maximum(jnp.minimum(ax2, d_x2)
                                 - jnp.maximum(ax1, d_x1), 0.0)
                     * jnp.maximum(jnp.minimum(ay2, d_y2)
                                   - jnp.maximum(ay1, d_y1), 0.0))
            union = jnp.maximum(area_a + area_b - inter, 1e-10)
            iou = jnp.where(valid, inter / union, -1.0)
            iou_scr[i, j] = iou
            upd = iou > best
            best = jnp.where(upd, iou, best)
            best_idx = jnp.where(upd, j, best_idx)
        bests.append(best)
        best_idxs.append(best_idx)

    allio = iou_scr[...]
    m_all = jnp.max(allio, axis=(2, 3), keepdims=True)
    cand = jnp.where(allio == m_all, fi[None, None], jnp.int32(1 << 30))
    bi = jnp.min(cand, axis=(2, 3), keepdims=True)

    for i in range(ipb):
        best = bests[i]
        best_idx = best_idxs[i]
        for j in range(n_obj):
            best_idx = jnp.where(fi == bi[i, j], j, best_idx)

        check = best > 0.5

        g_lab = jnp.zeros((S, L), jnp.int32)
        g_cx = jnp.zeros((S, L), jnp.float32)
        g_cy = jnp.zeros((S, L), jnp.float32)
        g_w = jnp.zeros((S, L), jnp.float32)
        g_h = jnp.zeros((S, L), jnp.float32)
        for j in range(n_obj):
            eq = best_idx == j
            cx, cy, w, h = tbs[i][j]
            g_lab = jnp.where(eq, tls[i][j], g_lab)
            g_cx = jnp.where(eq, cx, g_cx)
            g_cy = jnp.where(eq, cy, g_cy)
            g_w = jnp.where(eq, w, g_w)
            g_h = jnp.where(eq, h, g_h)

        true_lab = jnp.where(check, g_lab, 0)
        positive = true_lab != 0
        t_cx = jnp.where(check, g_cx, 0.0)
        t_cy = jnp.where(check, g_cy, 0.0)
        t_w = jnp.where(check, g_w, 0.0)
        t_h = jnp.where(check, g_h, 0.0)

        e_x = (t_cx - p_cx) / (p_w * 0.1)
        e_y = (t_cy - p_cy) / (p_h * 0.1)
        e_w = jnp.log(jnp.maximum(t_w, 1e-8) / p_w) * 5.0
        e_h = jnp.log(jnp.maximum(t_h, 1e-8) / p_h) * 5.0

        hub = jnp.zeros((S, L), jnp.float32)
        for c, enc in enumerate((e_x, e_y, e_w, e_h)):
            diff = jnp.abs(pb_ref[i, c] - enc)
            hub = hub + jnp.where(diff < 1.0, 0.5 * diff * diff, diff - 0.5)
        hub_sum = jnp.sum(jnp.where(positive, hub, 0.0))

        mx = sc_ref[i, 0]
        for c in range(1, n_classes):
            mx = jnp.maximum(mx, sc_ref[i, c])
        ssum = jnp.zeros((S, L), jnp.float32)
        sel = jnp.zeros((S, L), jnp.float32)
        for c in range(n_classes):
            v = sc_ref[i, c]
            ssum = ssum + jnp.exp(v - mx)
            sel = jnp.where(true_lab == c, v, sel)
        ce = mx + jnp.log(ssum) - sel

        n_pos = jnp.sum(jnp.where(positive, 1.0, 0.0))
        pce = jnp.sum(jnp.where(positive, ce, 0.0))
        neg = jnp.where(valid, jnp.where(positive, 0.0, ce), -1.0)

        neg_ref[i] = neg
        npos_ref[i] = jnp.full((1, L), n_pos, jnp.float32)
        hub_ref[i] = jnp.full((1, L), hub_sum, jnp.float32)
        pce_ref[i] = jnp.full((1, L), pce, jnp.float32)


def _mine_kernel(neg_ref, npos_ref, hub_ref, pce_ref, out_ref):
    neg = neg_ref[...]
    bits = lax.bitcast_convert_type(neg, jnp.int32)
    k = npos_ref[:, :, 0:1] * 3.0

    def body(_, carry):
        lo, hi = carry
        mid = lo + ((hi - lo + 1) >> 1)
        cnt = jnp.sum(jnp.where(bits >= mid, 1.0, 0.0), axis=(1, 2),
                      keepdims=True)
        pred = cnt >= k
        return (jnp.where(pred, mid, lo),
                jnp.where(pred, hi, mid - 1))

    B = neg.shape[0]
    lo0 = jnp.zeros((B, 1, 1), jnp.int32)
    hi0 = jnp.full((B, 1, 1), _INF_BITS, jnp.int32)
    lo, _ = lax.fori_loop(0, 31, body, (lo0, hi0))
    t = lax.bitcast_convert_type(lo, jnp.float32)

    gt = neg > t
    cnt_gt = jnp.sum(jnp.where(gt, 1.0, 0.0), axis=(1, 2), keepdims=True)
    sum_gt = jnp.sum(jnp.where(gt, neg, 0.0), axis=(1, 2), keepdims=True)
    hn_row = sum_gt + t * (k - cnt_gt)
    hn = jnp.sum(jnp.where(k > 0.0, hn_row, 0.0))

    n_pos = jnp.sum(npos_ref[:, :, 0:1])
    pos_ce = jnp.sum(pce_ref[:, :, 0:1])
    hub = jnp.sum(hub_ref[:, :, 0:1])
    loss = (pos_ce + hn) / n_pos + hub / (4.0 * n_pos)
    out_ref[...] = jnp.full((8, _LANES), loss, jnp.float32)


@jax.jit
def kernel(predicted_boxes, predicted_scores, target_boxes, target_labels,
           default_boxes):
    B, N, C = predicted_scores.shape
    n_obj = target_boxes.shape[1]
    L = _LANES
    S = -(-N // L)
    S = -(-S // 8) * 8
    pad = S * L - N
    ipb = 2
    while B % ipb:
        ipb = 1

    ps = jnp.pad(jnp.transpose(predicted_scores, (0, 2, 1)),
                 ((0, 0), (0, 0), (0, pad))).reshape(B, C, S, L)
    pb = jnp.pad(jnp.transpose(predicted_boxes, (0, 2, 1)),
                 ((0, 0), (0, 0), (0, pad))).reshape(B, 4, S, L)
    db = jnp.pad(jnp.transpose(default_boxes, (1, 0)),
                 ((0, 0), (0, pad))).reshape(4, S, L)
    tb = target_boxes.astype(jnp.float32)
    tl = target_labels.astype(jnp.int32).reshape(B, 1, n_obj)

    f32 = jnp.float32
    neg, npos, hub, pce = pl.pallas_call(
        functools.partial(_image_kernel, n_obj, C, N, ipb),
        grid=(B // ipb,),
        in_specs=[
            pl.BlockSpec((ipb, C, S, L), lambda b: (b, 0, 0, 0)),
            pl.BlockSpec((ipb, 4, S, L), lambda b: (b, 0, 0, 0)),
            pl.BlockSpec((ipb, n_obj, 4), lambda b: (b, 0, 0)),
            pl.BlockSpec((ipb, 1, n_obj), lambda b: (b, 0, 0)),
            pl.BlockSpec((4, S, L), lambda b: (0, 0, 0)),
        ],
        out_specs=[
            pl.BlockSpec((ipb, S, L), lambda b: (b, 0, 0)),
            pl.BlockSpec((ipb, 1, L), lambda b: (b, 0, 0)),
            pl.BlockSpec((ipb, 1, L), lambda b: (b, 0, 0)),
            pl.BlockSpec((ipb, 1, L), lambda b: (b, 0, 0)),
        ],
        out_shape=[
            jax.ShapeDtypeStruct((B, S, L), f32),
            jax.ShapeDtypeStruct((B, 1, L), f32),
            jax.ShapeDtypeStruct((B, 1, L), f32),
            jax.ShapeDtypeStruct((B, 1, L), f32),
        ],
        scratch_shapes=[pltpu.VMEM((ipb, n_obj, S, L), f32)],
    )(ps, pb, tb, tl, db)

    out = pl.pallas_call(
        _mine_kernel,
        out_shape=jax.ShapeDtypeStruct((8, L), f32),
    )(neg, npos, hub, pce)
    return out[0, 0]
